# Initial kernel scaffold; baseline (speedup 1.0000x reference)
import jax
import jax.numpy as jnp
from jax.experimental import pallas as pl


def kernel(w1, b1, w2, b2, w3, b3, x, t):
    raise NotImplementedError("write your pallas kernel here")


# speedup vs baseline: 2.9846x; 2.9846x over previous
import functools

import jax
import jax.numpy as jnp
from jax import lax
from jax.experimental import pallas as pl
from jax.experimental.pallas import tpu as pltpu


def _denoiser_kernel(t_ref, x_ref, w1_ref, b1_ref, w2_ref, b2_ref,
                     w3_ref, b3_ref, o_ref, in_buf, copy_sem, *,
                     H, W, TH, Cin, Ch):
    b = pl.program_id(0)
    rt = pl.program_id(1)
    r0 = rt * TH

    start = pl.multiple_of(r0, 8)
    cp = pltpu.make_async_copy(x_ref.at[b, pl.ds(start, TH + 6), :, :],
                               in_buf, copy_sem)
    cp.start()
    cp.wait()

    R0, R1, R2 = TH + 6, TH + 4, TH + 2
    cd = jnp.bfloat16

    def row_mask(nrows, first_global_row):
        g = lax.broadcasted_iota(jnp.int32, (nrows, 1, 1), 0) + first_global_row
        return (g >= 0) & (g < H)

    def pad_cols(h):
        z = jnp.zeros((h.shape[0], 1, h.shape[2]), h.dtype)
        return jnp.concatenate([z, h, z], axis=1)

    t_val = t_ref[b]
    t_full = jnp.where(row_mask(R0, r0 - 3), t_val, 0.0).astype(cd)
    t_full = jnp.broadcast_to(t_full, (R0, W, 1))
    x_in = jnp.concatenate([in_buf[...], t_full], axis=-1)
    xp = pad_cols(x_in)
    lhs1 = jnp.concatenate(
        [xp[dy:dy + R1, dx:dx + W, :] for dy in range(3) for dx in range(3)],
        axis=-1)
    a1 = jnp.dot(lhs1.reshape(R1 * W, 9 * (Cin + 1)), w1_ref[...],
                 preferred_element_type=jnp.float32)
    a1 = jnp.maximum(a1 + b1_ref[...], 0.0).reshape(R1, W, Ch)
    h1 = jnp.where(row_mask(R1, r0 - 2), a1, 0.0).astype(cd)

    h1p = pad_cols(h1)
    cc2 = jnp.concatenate([h1p[:, dx:dx + W, :] for dx in range(3)], axis=-1)
    acc2 = jnp.dot(cc2[0:R2].reshape(R2 * W, 3 * Ch), w2_ref[0],
                   preferred_element_type=jnp.float32)
    for dy in range(1, 3):
        acc2 = acc2 + jnp.dot(cc2[dy:dy + R2].reshape(R2 * W, 3 * Ch),
                              w2_ref[dy], preferred_element_type=jnp.float32)
    a2 = jnp.maximum(acc2 + b2_ref[...], 0.0).reshape(R2, W, Ch)
    h2 = jnp.where(row_mask(R2, r0 - 1), a2, 0.0).astype(cd)

    h2p = pad_cols(h2)
    cc3 = jnp.concatenate([h2p[:, dx:dx + W, :] for dx in range(3)], axis=-1)
    acc3 = jnp.dot(cc3[0:TH].reshape(TH * W, 3 * Ch), w3_ref[0],
                   preferred_element_type=jnp.float32)
    for dy in range(1, 3):
        acc3 = acc3 + jnp.dot(cc3[dy:dy + TH].reshape(TH * W, 3 * Ch),
                              w3_ref[dy], preferred_element_type=jnp.float32)
    o_ref[0] = jnp.transpose(acc3) + b3_ref[...]


def _pick_tile_rows(H):
    for th in (32, 16, 8):
        if th <= H and H % th == 0:
            return th
    return H


def kernel(w1, b1, w2, b2, w3, b3, x, t):
    B, Cin, H, W = x.shape
    Ch = w1.shape[-1]
    C0 = Cin + 1
    TH = _pick_tile_rows(H)

    cd = jnp.bfloat16
    xh = jnp.transpose(x, (0, 2, 3, 1)).astype(cd)
    x_pad = jnp.pad(xh, ((0, 0), (3, 3), (0, 0), (0, 0)))
    t_norm = t.astype(jnp.float32) / 1000.0

    w1r = w1.reshape(9 * C0, Ch).astype(cd)
    w2r = w2.reshape(3, 3 * Ch, Ch).astype(cd)
    w3r = w3.reshape(3, 3 * Ch, Cin).astype(cd)
    b1r = b1.reshape(1, Ch).astype(jnp.float32)
    b2r = b2.reshape(1, Ch).astype(jnp.float32)
    b3r = b3.reshape(Cin, 1).astype(jnp.float32)

    kern = functools.partial(_denoiser_kernel, H=H, W=W, TH=TH, Cin=Cin, Ch=Ch)

    flops = 2 * B * H * W * 9 * (C0 * Ch + Ch * Ch + Ch * Cin)
    bytes_accessed = int(B * (H + 6) * W * Cin * 2 + B * H * W * Cin * 4
                         + 9 * (C0 * Ch + Ch * Ch + Cin * Ch) * 2)
    cost = pl.CostEstimate(flops=flops, transcendentals=0,
                           bytes_accessed=bytes_accessed)

    out = pl.pallas_call(
        kern,
        out_shape=jax.ShapeDtypeStruct((B, Cin, H * W), jnp.float32),
        grid_spec=pltpu.PrefetchScalarGridSpec(
            num_scalar_prefetch=0,
            grid=(B, H // TH),
            in_specs=[
                pl.BlockSpec(memory_space=pltpu.MemorySpace.SMEM),
                pl.BlockSpec(memory_space=pl.ANY),
                pl.BlockSpec((9 * C0, Ch), lambda b, r: (0, 0)),
                pl.BlockSpec((1, Ch), lambda b, r: (0, 0)),
                pl.BlockSpec((3, 3 * Ch, Ch), lambda b, r: (0, 0, 0)),
                pl.BlockSpec((1, Ch), lambda b, r: (0, 0)),
                pl.BlockSpec((3, 3 * Ch, Cin), lambda b, r: (0, 0, 0)),
                pl.BlockSpec((Cin, 1), lambda b, r: (0, 0)),
            ],
            out_specs=pl.BlockSpec((1, Cin, TH * W), lambda b, r: (b, 0, r)),
            scratch_shapes=[
                pltpu.VMEM((TH + 6, W, Cin), cd),
                pltpu.SemaphoreType.DMA,
            ],
        ),
        compiler_params=pltpu.CompilerParams(
            dimension_semantics=("parallel", "parallel"),
            vmem_limit_bytes=56 * 2 ** 20,
        ),
        cost_estimate=cost,
    )(t_norm, x_pad, w1r, b1r, w2r, b2r, w3r, b3r)

    return out.reshape(B, Cin, H, W)


# speedup vs baseline: 3.4629x; 1.1603x over previous
import functools

import jax
import jax.numpy as jnp
from jax import lax
from jax.experimental import pallas as pl
from jax.experimental.pallas import tpu as pltpu


def _denoiser_kernel(t_ref, x_ref, w1_ref, b1_ref, w2_ref, b2_ref,
                     w3_ref, b3_ref, o_ref, in_buf, copy_sem, *,
                     H, W, TH, Cin, Ch):
    b = pl.program_id(0)
    rt = pl.program_id(1)
    r0 = rt * TH

    start = pl.multiple_of(r0, 8)
    cp = pltpu.make_async_copy(x_ref.at[b, pl.ds(start, TH + 6), :, :],
                               in_buf, copy_sem)
    cp.start()
    cp.wait()

    R0, R1, R2 = TH + 6, TH + 4, TH + 2
    cd = jnp.bfloat16

    def row_mask(nrows, first_global_row):
        g = lax.broadcasted_iota(jnp.int32, (nrows, 1, 1), 0) + first_global_row
        return (g >= 0) & (g < H)

    def colcat(h):
        z = jnp.zeros((h.shape[0], 1, h.shape[2]), h.dtype)
        right = jnp.concatenate([z, h[:, :W - 1, :]], axis=1)
        left = jnp.concatenate([h[:, 1:, :], z], axis=1)
        return jnp.concatenate([right, h, left], axis=-1)

    def conv3dots(cc, w_ref, nrows_out):
        kdim = cc.shape[-1]
        acc = jnp.dot(cc[0:nrows_out].reshape(nrows_out * W, kdim), w_ref[0],
                      preferred_element_type=jnp.float32)
        for dy in range(1, 3):
            acc = acc + jnp.dot(
                cc[dy:dy + nrows_out].reshape(nrows_out * W, kdim),
                w_ref[dy], preferred_element_type=jnp.float32)
        return acc

    t_val = t_ref[b]
    t_col = jnp.where(row_mask(R0, r0 - 3), t_val, 0.0).astype(cd)
    t_full = jnp.broadcast_to(t_col, (R0, W, 1))
    x_in = jnp.concatenate([in_buf[...], t_full], axis=-1)
    a1 = conv3dots(colcat(x_in), w1_ref, R1)
    a1 = jnp.maximum(a1 + b1_ref[...], 0.0).reshape(R1, W, Ch)
    h1 = jnp.where(row_mask(R1, r0 - 2), a1, 0.0).astype(cd)

    acc2 = conv3dots(colcat(h1), w2_ref, R2)
    a2 = jnp.maximum(acc2 + b2_ref[...], 0.0).reshape(R2, W, Ch)
    h2 = jnp.where(row_mask(R2, r0 - 1), a2, 0.0).astype(cd)

    acc3 = conv3dots(colcat(h2), w3_ref, TH)
    o_ref[0] = jnp.transpose(acc3) + b3_ref[...]


def _pick_tile_rows(H):
    for th in (64, 32, 16, 8):
        if th <= H and H % th == 0:
            return th
    return H


def kernel(w1, b1, w2, b2, w3, b3, x, t):
    B, Cin, H, W = x.shape
    Ch = w1.shape[-1]
    C0 = Cin + 1
    TH = _pick_tile_rows(H)

    cd = jnp.bfloat16
    xh = jnp.transpose(x, (0, 2, 3, 1)).astype(cd)
    x_pad = jnp.pad(xh, ((0, 0), (3, 3), (0, 0), (0, 0)))
    t_norm = t.astype(jnp.float32) / 1000.0

    w1r = w1.reshape(3, 3 * C0, Ch).astype(cd)
    w2r = w2.reshape(3, 3 * Ch, Ch).astype(cd)
    w3r = w3.reshape(3, 3 * Ch, Cin).astype(cd)
    b1r = b1.reshape(1, Ch).astype(jnp.float32)
    b2r = b2.reshape(1, Ch).astype(jnp.float32)
    b3r = b3.reshape(Cin, 1).astype(jnp.float32)

    kern = functools.partial(_denoiser_kernel, H=H, W=W, TH=TH, Cin=Cin, Ch=Ch)

    flops = 2 * B * H * W * 9 * (C0 * Ch + Ch * Ch + Ch * Cin)
    bytes_accessed = int(B * (H + 6) * W * Cin * 2 + B * H * W * Cin * 4
                         + 9 * (C0 * Ch + Ch * Ch + Cin * Ch) * 2)
    cost = pl.CostEstimate(flops=flops, transcendentals=0,
                           bytes_accessed=bytes_accessed)

    out = pl.pallas_call(
        kern,
        out_shape=jax.ShapeDtypeStruct((B, Cin, H * W), jnp.float32),
        grid_spec=pltpu.PrefetchScalarGridSpec(
            num_scalar_prefetch=0,
            grid=(B, H // TH),
            in_specs=[
                pl.BlockSpec(memory_space=pltpu.MemorySpace.SMEM),
                pl.BlockSpec(memory_space=pl.ANY),
                pl.BlockSpec((3, 3 * C0, Ch), lambda b, r: (0, 0, 0)),
                pl.BlockSpec((1, Ch), lambda b, r: (0, 0)),
                pl.BlockSpec((3, 3 * Ch, Ch), lambda b, r: (0, 0, 0)),
                pl.BlockSpec((1, Ch), lambda b, r: (0, 0)),
                pl.BlockSpec((3, 3 * Ch, Cin), lambda b, r: (0, 0, 0)),
                pl.BlockSpec((Cin, 1), lambda b, r: (0, 0)),
            ],
            out_specs=pl.BlockSpec((1, Cin, TH * W), lambda b, r: (b, 0, r)),
            scratch_shapes=[
                pltpu.VMEM((TH + 6, W, Cin), cd),
                pltpu.SemaphoreType.DMA,
            ],
        ),
        compiler_params=pltpu.CompilerParams(
            dimension_semantics=("parallel", "parallel"),
            vmem_limit_bytes=56 * 2 ** 20,
        ),
        cost_estimate=cost,
    )(t_norm, x_pad, w1r, b1r, w2r, b2r, w3r, b3r)

    return out.reshape(B, Cin, H, W)
